# baseline (speedup 1.0000x reference)
import jax
import jax.numpy as jnp
from jax.experimental import pallas as pl

N = 10000
E = 320000
R = 3
H = 4
DH = 32


def _layer(x, src, dst, et, W, al, ar, act_inner, act_outer):
    xW = jnp.einsum('nd,rdk->rnk', x, W).reshape(R, N, H, DH)
    el = (xW * al[:, None, :, :]).sum(-1)
    er = (xW * ar[:, None, :, :]).sum(-1)
    e = el[et, src] + er[et, dst]
    e = jax.nn.leaky_relu(e, negative_slope=0.2)
    seg = dst * R + et
    nseg = N * R
    m = jax.ops.segment_max(e, seg, num_segments=nseg)
    m = jnp.where(jnp.isfinite(m), m, 0.0)
    ex = jnp.exp(e - m[seg])
    denom = jax.ops.segment_sum(ex, seg, num_segments=nseg)
    alpha = ex / (denom[seg] + 1e-9)
    msg = xW[et, src] * alpha[..., None]
    out = jax.ops.segment_sum(msg, seg, num_segments=nseg).reshape(N, R, H, DH)
    if act_inner:
        out = jax.nn.relu(out)
    out = out.sum(axis=1).reshape(N, H * DH)
    if act_outer:
        out = jax.nn.relu(out)
    return out


def _identity_kernel(x_ref, o_ref):
    o_ref[...] = x_ref[...]


def kernel(x, edge_index, edge_type, W1, al1, ar1, W2, al2, ar2):
    src = edge_index[0]
    dst = edge_index[1]
    h = _layer(x, src, dst, edge_type, W1, al1, ar1, True, True)
    out = _layer(h, src, dst, edge_type, W2, al2, ar2, False, False)
    return pl.pallas_call(
        _identity_kernel,
        out_shape=jax.ShapeDtypeStruct(out.shape, out.dtype),
    )(out)


# speedup vs baseline: 1.1389x; 1.1389x over previous
import functools

import jax
import jax.numpy as jnp
from jax.experimental import pallas as pl
from jax.experimental.pallas import tpu as pltpu

N = 10000
E = 320000
R = 3
D = 128
H = 4
DH = 32

CHUNK = 6400
NSEG = N * R

_SMEM = getattr(pltpu, "SMEM", None)
if _SMEM is None:
    _SMEM = pltpu.MemorySpace.SMEM

_PARAMS = pltpu.CompilerParams(vmem_limit_bytes=120 * 1024 * 1024)


def _prep_kernel(x_ref, w_ref, ael_ref, aer_ref, xw_ref, m_ref):
    for r in range(R):
        xw = jnp.dot(x_ref[...], w_ref[r], preferred_element_type=jnp.float32)
        el = jnp.dot(xw, ael_ref[r], preferred_element_type=jnp.float32)
        er = jnp.dot(xw, aer_ref[r], preferred_element_type=jnp.float32)
        xw_ref[r * N:(r + 1) * N, :] = xw
        m = (jnp.max(el, axis=0, keepdims=True)
             + jnp.max(er, axis=0, keepdims=True))
        m_ref[r:r + 1, :] = jnp.where(m > 0, m, 0.2 * m)


def _edge_kernel(idx_ref, xw_ref, ael_ref, aer_ref, m_ref, u_ref, d_ref):
    @pl.when(pl.program_id(0) == 0)
    def _():
        u_ref[...] = jnp.zeros_like(u_ref)
        d_ref[...] = jnp.zeros_like(d_ref)

    def body(i, carry):
        s = idx_ref[0, i]
        t = idx_ref[1, i]
        r = idx_ref[2, i]
        row = xw_ref[pl.ds(s, 1), :]
        row_t = xw_ref[pl.ds(t, 1), :]
        alr = ael_ref[pl.ds(r, 1), :, :].reshape(D, D)
        arr = aer_ref[pl.ds(r, 1), :, :].reshape(D, D)
        elv = jnp.dot(row, alr, preferred_element_type=jnp.float32)
        erv = jnp.dot(row_t, arr, preferred_element_type=jnp.float32)
        mv = m_ref[pl.ds(r, 1), :]
        e = elv + erv
        e = jnp.where(e > 0, e, 0.2 * e)
        w = jnp.exp(e - mv)
        u_ref[pl.ds(t, 1), :] = u_ref[pl.ds(t, 1), :] + w * row
        d_ref[pl.ds(t, 1), :] = d_ref[pl.ds(t, 1), :] + w
        return carry

    jax.lax.fori_loop(0, CHUNK, body, 0)


def _final_kernel(u_ref, d_ref, o_ref, *, act_inner, act_outer):
    acc = jnp.zeros((N, 128), dtype=jnp.float32)
    for r in range(R):
        t = u_ref[r * N:(r + 1) * N, :] / (d_ref[r * N:(r + 1) * N, :] + 1e-30)
        if act_inner:
            t = jnp.maximum(t, 0.0)
        acc = acc + t
    if act_outer:
        acc = jnp.maximum(acc, 0.0)
    o_ref[...] = acc


def _layer(x, idx, W, AL, AR, act_inner, act_outer):
    f32 = jnp.float32
    xw, m = pl.pallas_call(
        _prep_kernel,
        out_shape=[
            jax.ShapeDtypeStruct((NSEG, 128), f32),
            jax.ShapeDtypeStruct((R, 128), f32),
        ],
        compiler_params=_PARAMS,
    )(x, W, AL, AR)

    grid = E // CHUNK
    u, d = pl.pallas_call(
        _edge_kernel,
        grid=(grid,),
        in_specs=[
            pl.BlockSpec((3, CHUNK), lambda i: (0, i), memory_space=_SMEM),
            pl.BlockSpec((NSEG, 128), lambda i: (0, 0)),
            pl.BlockSpec((R, D, D), lambda i: (0, 0, 0)),
            pl.BlockSpec((R, D, D), lambda i: (0, 0, 0)),
            pl.BlockSpec((R, 128), lambda i: (0, 0)),
        ],
        out_specs=[
            pl.BlockSpec((NSEG, 128), lambda i: (0, 0)),
            pl.BlockSpec((NSEG, 128), lambda i: (0, 0)),
        ],
        out_shape=[
            jax.ShapeDtypeStruct((NSEG, 128), f32),
            jax.ShapeDtypeStruct((NSEG, 128), f32),
        ],
        compiler_params=_PARAMS,
    )(idx, xw, AL, AR, m)

    return pl.pallas_call(
        functools.partial(_final_kernel, act_inner=act_inner,
                          act_outer=act_outer),
        out_shape=jax.ShapeDtypeStruct((N, 128), f32),
        compiler_params=_PARAMS,
    )(u, d)


def _logit_mats(a):
    eye = jnp.eye(H, dtype=a.dtype)
    blk = (a.reshape(R, H, DH, 1) * eye[None, :, None, :]).reshape(R, H * DH, H)
    rep = jnp.repeat(eye, DH, axis=1)
    return jnp.einsum('rkh,hl->rkl', blk, rep)


def kernel(x, edge_index, edge_type, W1, al1, ar1, W2, al2, ar2):
    src = edge_index[0].astype(jnp.int32)
    dst = edge_index[1].astype(jnp.int32)
    et = edge_type.astype(jnp.int32)
    idx = jnp.stack([et * N + src, et * N + dst, et], axis=0)
    h = _layer(x, idx, W1, _logit_mats(al1), _logit_mats(ar1), True, True)
    return _layer(h, idx, W2, _logit_mats(al2), _logit_mats(ar2), False, False)
